# Initial kernel scaffold; baseline (speedup 1.0000x reference)
import jax
import jax.numpy as jnp
from jax.experimental import pallas as pl


def kernel(pred, target, indices, indices_type, type_names):
    raise NotImplementedError("write your pallas kernel here")


# speedup vs baseline: 1.5128x; 1.5128x over previous
import functools

import jax
import jax.numpy as jnp
from jax.experimental import pallas as pl
from jax.experimental.pallas import tpu as pltpu


def _body(pred_ref, target_ref, s_ref, e_ref, it_ref, out_ref, acc_ref,
          *, B, N, D, G, T):
    b = pl.program_id(0)
    p = pred_ref[0]
    t = target_ref[0]
    d = p - t
    err = d * d
    ones_d = jnp.ones((D, 1), jnp.float32)
    tok = jax.lax.dot_general(
        err, ones_d, (((1,), (0,)), ((), ())),
        preferred_element_type=jnp.float32,
        precision=jax.lax.Precision.HIGHEST) * (1.0 / D)

    s = s_ref[0]
    e = e_ref[0]
    n_iota = jax.lax.broadcasted_iota(jnp.int32, (G, N), 1)
    mask = ((n_iota >= s) & (n_iota < e)).astype(jnp.float32)
    gsum = jax.lax.dot_general(
        mask, tok, (((1,), (0,)), ((), ())),
        preferred_element_type=jnp.float32,
        precision=jax.lax.Precision.HIGHEST)
    cnt = (e - s).astype(jnp.float32)
    g_err = gsum / jnp.maximum(cnt, 1.0)

    it = it_ref[0]
    maxv = jnp.max(it, axis=1, keepdims=True)
    colidx = jax.lax.broadcasted_iota(jnp.int32, (G, T), 1)
    am = jnp.min(jnp.where(it == maxv, colidx, T), axis=1, keepdims=True)
    onehot = (colidx == am).astype(jnp.float32)

    tsum = jnp.sum(onehot * g_err, axis=0, keepdims=True)
    tcnt = jnp.sum(onehot, axis=0, keepdims=True)
    part = jnp.concatenate([tsum, tcnt], axis=0)

    @pl.when(b == 0)
    def _():
        acc_ref[0:2, 0:T] = part

    @pl.when(b > 0)
    def _():
        acc_ref[0:2, 0:T] = acc_ref[0:2, 0:T] + part

    @pl.when(b == B - 1)
    def _():
        vals = acc_ref[0:2, 0:T]
        ts = vals[0:1, :]
        tc = vals[1:2, :]
        out_ref[...] = jnp.where(tc > 0, ts / jnp.maximum(tc, 1.0), 0.0)


def kernel(pred, target, indices, indices_type, type_names):
    B, N, D = pred.shape
    G = indices.shape[1] - 1
    T = indices_type.shape[2]
    starts = indices[:, :-1, None]
    ends = indices[:, 1:, None]
    out = pl.pallas_call(
        functools.partial(_body, B=B, N=N, D=D, G=G, T=T),
        grid=(B,),
        in_specs=[
            pl.BlockSpec((1, N, D), lambda b: (b, 0, 0)),
            pl.BlockSpec((1, N, D), lambda b: (b, 0, 0)),
            pl.BlockSpec((1, G, 1), lambda b: (b, 0, 0)),
            pl.BlockSpec((1, G, 1), lambda b: (b, 0, 0)),
            pl.BlockSpec((1, G, T), lambda b: (b, 0, 0)),
        ],
        out_specs=pl.BlockSpec((1, T), lambda b: (0, 0)),
        out_shape=jax.ShapeDtypeStruct((1, T), jnp.float32),
        scratch_shapes=[pltpu.VMEM((8, 128), jnp.float32)],
    )(pred, target, starts, ends, indices_type)
    return out.reshape(T)
